# baseline (speedup 1.0000x reference)
import jax
import jax.numpy as jnp
from jax.experimental import pallas as pl
from jax.experimental.pallas import tpu as pltpu

_D = 4096
_H = 2 * _D
_E = 64
_K = 8
_N = 8192

_TBLK = 512
_JBLK = 256
_NJ = _H // _JBLK
_CBLK = 128
_NC = (_D // 2) // _CBLK
_UBLK = 512
_PBLK = 256

_F32 = jnp.float32


def _head_kernel(x_ref, w1_ref, w2_ref, wc1_ref, w3_ref,
                 b1_ref, b2_ref, b3_ref, bc1_ref, wc2_ref, bc2_ref,
                 logits_ref, tkl_ref, tki_ref, conf_ref,
                 acc_ref):
    j = pl.program_id(1)
    nj = pl.num_programs(1)

    @pl.when(j == 0)
    def _init():
        conf_ref[...] = jnp.zeros_like(conf_ref)

    x = x_ref[...]
    h1 = jnp.maximum(
        jnp.dot(x, w1_ref[...], preferred_element_type=_F32) + b1_ref[...],
        0.0)
    for k in range(_D // _UBLK):
        sl = slice(k * _UBLK, (k + 1) * _UBLK)
        upd = jnp.dot(h1, w2_ref[:, sl], preferred_element_type=_F32)
        if _NJ > 1:
            acc_ref[:, sl] = jnp.where(j == 0, upd, acc_ref[:, sl] + upd)
        else:
            acc_ref[:, sl] = upd

    @pl.when(j < _NC)
    def _conf_chunk():
        c = jnp.maximum(
            jnp.dot(x, wc1_ref[...], preferred_element_type=_F32)
            + bc1_ref[...], 0.0)
        conf_ref[...] += jnp.sum(c * wc2_ref[...], axis=1, keepdims=True)

    @pl.when(j == nj - 1)
    def _epilogue():
        logits = b3_ref[...] + jnp.zeros((_TBLK, _E), _F32)
        for k in range(_D // _PBLK):
            sl = slice(k * _PBLK, (k + 1) * _PBLK)
            h2 = jnp.maximum(acc_ref[:, sl] + b2_ref[:, sl], 0.0)
            logits = logits + jnp.dot(h2, w3_ref[sl, :],
                                      preferred_element_type=_F32)
        logits_ref[...] = logits
        conf_ref[...] = jax.nn.sigmoid(conf_ref[...] + bc2_ref[0, 0])

        iota = jax.lax.broadcasted_iota(jnp.int32, logits.shape, 1)
        work = logits
        vals, idxs = [], []
        for _ in range(_K):
            m = jnp.max(work, axis=1, keepdims=True)
            idx = jnp.min(jnp.where(work == m, iota, _E), axis=1, keepdims=True)
            vals.append(m)
            idxs.append(idx)
            work = jnp.where(iota == idx, -jnp.inf, work)
        tkl_ref[...] = jnp.concatenate(vals, axis=1)
        tki_ref[...] = jnp.concatenate(idxs, axis=1)


def kernel(x, W1, b1, W2, b2, W3, b3, Wc1, bc1, Wc2, bc2):
    w1t = W1.T
    w2t = W2.T
    wc1t = Wc1.T
    w3t = W3.T
    b1r = b1.reshape(1, _H)
    b2r = b2.reshape(1, _D)
    b3r = b3.reshape(1, _E)
    bc1r = bc1.reshape(1, _D // 2)
    bc2r = bc2.reshape(1, 1)

    cclamp = lambda i, j: (jnp.minimum(j, _NC - 1), 0)
    cclamp_row = lambda i, j: (0, jnp.minimum(j, _NC - 1))
    grid = (_N // _TBLK, _NJ)
    outs = pl.pallas_call(
        _head_kernel,
        grid=grid,
        in_specs=[
            pl.BlockSpec((_TBLK, _D), lambda i, j: (i, 0)),
            pl.BlockSpec((_D, _JBLK), lambda i, j: (0, j)),
            pl.BlockSpec((_JBLK, _D), lambda i, j: (j, 0)),
            pl.BlockSpec((_D, _CBLK), cclamp_row),
            pl.BlockSpec((_D, _E), lambda i, j: (0, 0)),
            pl.BlockSpec((1, _JBLK), lambda i, j: (0, j)),
            pl.BlockSpec((1, _D), lambda i, j: (0, 0)),
            pl.BlockSpec((1, _E), lambda i, j: (0, 0)),
            pl.BlockSpec((1, _CBLK), cclamp_row),
            pl.BlockSpec((1, _CBLK), cclamp_row),
            pl.BlockSpec((1, 1), lambda i, j: (0, 0)),
        ],
        out_specs=[
            pl.BlockSpec((_TBLK, _E), lambda i, j: (i, 0)),
            pl.BlockSpec((_TBLK, _K), lambda i, j: (i, 0)),
            pl.BlockSpec((_TBLK, _K), lambda i, j: (i, 0)),
            pl.BlockSpec((_TBLK, 1), lambda i, j: (i, 0)),
        ],
        out_shape=[
            jax.ShapeDtypeStruct((_N, _E), _F32),
            jax.ShapeDtypeStruct((_N, _K), _F32),
            jax.ShapeDtypeStruct((_N, _K), jnp.int32),
            jax.ShapeDtypeStruct((_N, 1), _F32),
        ],
        scratch_shapes=[
            pltpu.VMEM((_TBLK, _D), _F32),
        ],
        compiler_params=pltpu.CompilerParams(
            dimension_semantics=("parallel", "arbitrary"),
        ),
    )(x, w1t, w2t, wc1t, w3t, b1r, b2r, b3r, bc1r, Wc2, bc2r)

    expert_logits, top_k_logits, top_k_indices, confidence = outs
    return (expert_logits, top_k_logits, top_k_indices, confidence)


# speedup vs baseline: 1.5886x; 1.5886x over previous
import jax
import jax.numpy as jnp
from jax.experimental import pallas as pl
from jax.experimental.pallas import tpu as pltpu

_D = 4096
_H = 2 * _D
_E = 64
_K = 8
_N = 8192

_TBLK = 512
_JBLK = 512
_NJ = _H // _JBLK
_CBLK = 128
_NC = (_D // 2) // _CBLK
_UBLK = 512
_PBLK = 256

_F32 = jnp.float32
_BF = jnp.bfloat16


def _head_kernel(x_ref, w1_ref, w2_ref, wc1_ref, w3_ref,
                 b1_ref, b2_ref, b3_ref, bc1_ref, wc2_ref, bc2_ref,
                 logits_ref, tkl_ref, tki_ref, conf_ref,
                 acc_ref):
    j = pl.program_id(1)
    nj = pl.num_programs(1)

    @pl.when(j == 0)
    def _init():
        conf_ref[...] = jnp.zeros_like(conf_ref)

    x = x_ref[...]
    h1 = jnp.maximum(
        jnp.dot(x, w1_ref[...], preferred_element_type=_F32) + b1_ref[...],
        0.0)
    h1b = h1.astype(_BF)
    for k in range(_D // _UBLK):
        sl = slice(k * _UBLK, (k + 1) * _UBLK)
        upd = jnp.dot(h1b, w2_ref[:, sl], preferred_element_type=_F32)
        if _NJ > 1:
            acc_ref[:, sl] = jnp.where(j == 0, upd, acc_ref[:, sl] + upd)
        else:
            acc_ref[:, sl] = upd

    @pl.when(j < _NC)
    def _conf_chunk():
        c = jnp.maximum(
            jnp.dot(x, wc1_ref[...], preferred_element_type=_F32)
            + bc1_ref[...], 0.0)
        conf_ref[...] += jnp.sum(c * wc2_ref[...], axis=1, keepdims=True)

    @pl.when(j == nj - 1)
    def _epilogue():
        logits = b3_ref[...] + jnp.zeros((_TBLK, _E), _F32)
        for k in range(_D // _PBLK):
            sl = slice(k * _PBLK, (k + 1) * _PBLK)
            h2 = jnp.maximum(acc_ref[:, sl] + b2_ref[:, sl], 0.0)
            logits = logits + jnp.dot(h2.astype(_BF), w3_ref[sl, :],
                                      preferred_element_type=_F32)
        logits_ref[...] = logits
        conf_ref[...] = jax.nn.sigmoid(conf_ref[...] + bc2_ref[0, 0])

        iota = jax.lax.broadcasted_iota(jnp.int32, logits.shape, 1)
        work = logits
        vals, idxs = [], []
        for _ in range(_K):
            m = jnp.max(work, axis=1, keepdims=True)
            idx = jnp.min(jnp.where(work == m, iota, _E), axis=1, keepdims=True)
            vals.append(m)
            idxs.append(idx)
            work = jnp.where(iota == idx, -jnp.inf, work)
        tkl_ref[...] = jnp.concatenate(vals, axis=1)
        tki_ref[...] = jnp.concatenate(idxs, axis=1)


def kernel(x, W1, b1, W2, b2, W3, b3, Wc1, bc1, Wc2, bc2):
    xb = x.astype(_BF)
    w1t = W1.T.astype(_BF)
    w2t = W2.T.astype(_BF)
    wc1t = Wc1.T.astype(_BF)
    w3t = W3.T.astype(_BF)
    b1r = b1.reshape(1, _H)
    b2r = b2.reshape(1, _D)
    b3r = b3.reshape(1, _E)
    bc1r = bc1.reshape(1, _D // 2)
    bc2r = bc2.reshape(1, 1)

    cclamp_row = lambda i, j: (0, jnp.minimum(j, _NC - 1))
    grid = (_N // _TBLK, _NJ)
    outs = pl.pallas_call(
        _head_kernel,
        grid=grid,
        in_specs=[
            pl.BlockSpec((_TBLK, _D), lambda i, j: (i, 0)),
            pl.BlockSpec((_D, _JBLK), lambda i, j: (0, j)),
            pl.BlockSpec((_JBLK, _D), lambda i, j: (j, 0)),
            pl.BlockSpec((_D, _CBLK), cclamp_row),
            pl.BlockSpec((_D, _E), lambda i, j: (0, 0)),
            pl.BlockSpec((1, _JBLK), lambda i, j: (0, j)),
            pl.BlockSpec((1, _D), lambda i, j: (0, 0)),
            pl.BlockSpec((1, _E), lambda i, j: (0, 0)),
            pl.BlockSpec((1, _CBLK), cclamp_row),
            pl.BlockSpec((1, _CBLK), cclamp_row),
            pl.BlockSpec((1, 1), lambda i, j: (0, 0)),
        ],
        out_specs=[
            pl.BlockSpec((_TBLK, _E), lambda i, j: (i, 0)),
            pl.BlockSpec((_TBLK, _K), lambda i, j: (i, 0)),
            pl.BlockSpec((_TBLK, _K), lambda i, j: (i, 0)),
            pl.BlockSpec((_TBLK, 1), lambda i, j: (i, 0)),
        ],
        out_shape=[
            jax.ShapeDtypeStruct((_N, _E), _F32),
            jax.ShapeDtypeStruct((_N, _K), _F32),
            jax.ShapeDtypeStruct((_N, _K), jnp.int32),
            jax.ShapeDtypeStruct((_N, 1), _F32),
        ],
        scratch_shapes=[
            pltpu.VMEM((_TBLK, _D), _F32),
        ],
        compiler_params=pltpu.CompilerParams(
            dimension_semantics=("arbitrary", "arbitrary"),
        ),
    )(xb, w1t, w2t, wc1t, w3t, b1r, b2r, b3r, bc1r, Wc2, bc2r)

    expert_logits, top_k_logits, top_k_indices, confidence = outs
    return (expert_logits, top_k_logits, top_k_indices, confidence)
